# baseline (speedup 1.0000x reference)
import functools

import jax
import jax.numpy as jnp
from jax import lax
from jax.experimental import pallas as pl
from jax.experimental.pallas import tpu as pltpu
from jax.experimental.pallas import tpu_sc as plsc

_NC = 2
_NS = 16
_L = 16
_NW = _NC * _NS

_B = 16384
_D = 64
_BPW = _B // _NW
_CHUNK = 128
_NCH = _BPW // _CHUNK
_GROUPS = _BPW // _L


def _build(interpret=False):
  mesh = plsc.VectorSubcoreMesh(
      core_axis_name="c", subcore_axis_name="s",
      num_cores=_NC, num_subcores=_NS)

  @functools.partial(
      pl.kernel,
      out_type=jax.ShapeDtypeStruct((_B,), jnp.float32),
      mesh=mesh,
      scratch_types=[
          pltpu.VMEM((_NCH, _CHUNK), jnp.int32),
          pltpu.VMEM((_NCH, _CHUNK), jnp.int32),
          pltpu.VMEM((_BPW, _D), jnp.float32),
          pltpu.VMEM((_BPW, _D), jnp.float32),
          pltpu.VMEM((_BPW,), jnp.float32),
          pltpu.SemaphoreType.DMA,
      ],
      compiler_params=pltpu.CompilerParams(
          needs_layout_passes=False, use_tc_tiling_on_sc=False),
      interpret=interpret,
  )
  def mf(u_hbm, i_hbm, U_hbm, V_hbm, out_hbm,
         uidx_v, iidx_v, urows_v, vrows_v, out_v, sem):
    wid = lax.axis_index("s") * _NC + lax.axis_index("c")
    base = wid * _BPW

    pltpu.sync_copy(u_hbm.at[pl.ds(wid * _NCH, _NCH)], uidx_v)
    pltpu.sync_copy(i_hbm.at[pl.ds(wid * _NCH, _NCH)], iidx_v)

    copies = []
    for j in range(_NCH):
      copies.append(pltpu.async_copy(
          U_hbm.at[uidx_v.at[j]],
          urows_v.at[pl.ds(j * _CHUNK, _CHUNK)], sem))
      copies.append(pltpu.async_copy(
          V_hbm.at[iidx_v.at[j]],
          vrows_v.at[pl.ds(j * _CHUNK, _CHUNK)], sem))
    for c in copies:
      c.wait()

    iota = lax.broadcasted_iota(jnp.int32, (_L,), 0)

    def body(g, carry):
      r = g * _L + iota
      acc = jnp.zeros((_L,), jnp.float32)
      for d in range(_D):
        col = jnp.full((_L,), d, jnp.int32)
        ug = plsc.load_gather(urows_v, [r, col])
        vg = plsc.load_gather(vrows_v, [r, col])
        acc = acc + ug * vg
      out_v[pl.ds(g * _L, _L)] = acc
      return carry

    lax.fori_loop(0, _GROUPS, body, 0)

    pltpu.sync_copy(out_v, out_hbm.at[pl.ds(base, _BPW)])

  return mf


_mf = functools.cache(_build)


def kernel(u, i, U_emb, V_emb):
  u2 = u.astype(jnp.int32).reshape(_B // _CHUNK, _CHUNK)
  i2 = i.astype(jnp.int32).reshape(_B // _CHUNK, _CHUNK)
  return _mf()(u2, i2, U_emb, V_emb)
